# baseline (speedup 1.0000x reference)
import functools

import jax
import jax.numpy as jnp
from jax import lax
from jax.experimental import pallas as pl
from jax.experimental.pallas import tpu as pltpu
from jax.experimental.pallas import tpu_sc as plsc

K = 64


def _enc_body(x_ref, w_ref, be_ref, out_ref):
    acc = jax.lax.dot_general(
        x_ref[...], w_ref[...],
        dimension_numbers=(((1,), (1,)), ((), ())),
        preferred_element_type=jnp.float32,
    )
    out_ref[...] = acc + be_ref[...][None, :]


def _encode(x, W, b_enc, br, bl):
    n, d_in = x.shape
    d_lat = W.shape[0]
    grid = (d_lat // bl, n // br)
    return pl.pallas_call(
        _enc_body,
        grid=grid,
        in_specs=[
            pl.BlockSpec((br, d_in), lambda i, j: (j, 0)),
            pl.BlockSpec((bl, d_in), lambda i, j: (i, 0)),
            pl.BlockSpec((bl,), lambda i, j: (i,)),
        ],
        out_specs=pl.BlockSpec((br, bl), lambda i, j: (j, i)),
        out_shape=jax.ShapeDtypeStruct((n, d_lat), jnp.float32),
    )(x, W, b_enc)


def _thresh_body(pre_ref, t_ref):
    q = jnp.maximum(pre_ref[...], 0.0)
    s = jax.lax.bitcast_convert_type(q, jnp.int32)
    t = jnp.zeros((q.shape[0], 1), dtype=jnp.int32)
    for b in range(30, -1, -1):
        cand = t | (1 << b)
        cnt = jnp.sum(s >= cand, axis=1, keepdims=True, dtype=jnp.int32)
        t = jnp.where(cnt >= K, cand, t)
    t_ref[...] = jnp.broadcast_to(jnp.maximum(t, 1), (t.shape[0], 16))


def _thresholds(pre, br):
    n, d_lat = pre.shape
    return pl.pallas_call(
        _thresh_body,
        grid=(n // br,),
        in_specs=[pl.BlockSpec((br, d_lat), lambda i: (i, 0))],
        out_specs=pl.BlockSpec((br, 16), lambda i: (i, 0)),
        out_shape=jax.ShapeDtypeStruct((n, 16), jnp.int32),
    )(pre)


_GROUP = 2


def _sc_z(pre, t):
    n, d = pre.shape
    nw = 32
    rows_per_w = n // nw
    groups = rows_per_w // _GROUP
    mesh = plsc.VectorSubcoreMesh(core_axis_name="c", subcore_axis_name="s")

    @functools.partial(
        pl.kernel,
        mesh=mesh,
        out_type=jax.ShapeDtypeStruct((n, d), jnp.float32),
        scratch_types=[
            pltpu.VMEM((_GROUP, d), jnp.float32),
            pltpu.VMEM((_GROUP, d), jnp.float32),
            pltpu.VMEM((rows_per_w, 16), jnp.int32),
            pltpu.SemaphoreType.DMA,
            pltpu.SemaphoreType.DMA,
            pltpu.SemaphoreType.DMA,
            pltpu.SemaphoreType.DMA,
        ],
    )
    def zkern(pre_hbm, t_hbm, z_hbm, buf_a, buf_b, t_v, ia, ib, oa, ob):
        cid = lax.axis_index("c")
        sid = lax.axis_index("s")
        wid = sid * 2 + cid
        base = wid * rows_per_w
        pltpu.sync_copy(t_hbm.at[pl.ds(base, rows_per_w)], t_v)

        def src(g):
            return pre_hbm.at[pl.ds(base + g * _GROUP, _GROUP)]

        def dst(g):
            return z_hbm.at[pl.ds(base + g * _GROUP, _GROUP)]

        pltpu.async_copy(src(0), buf_a, ia)

        def step(g, buf, sem_in, sem_in_next, sem_out, sem_out_next, buf_other):
            pltpu.make_async_copy(src(g), buf, sem_in).wait()

            @pl.when(g + 1 < groups)
            def _():
                @pl.when(g >= 1)
                def _():
                    pltpu.make_async_copy(buf_other, dst(g - 1), sem_out_next).wait()
                pltpu.async_copy(src(g + 1), buf_other, sem_in_next)

            for r in range(_GROUP):
                tvec = t_v[g * _GROUP + r, :]

                def chunk(j, _, r=r, tvec=tvec):
                    for u in range(8):
                        sl = pl.ds(j * 128 + u * 16, 16)
                        v = buf[r, sl]
                        q = jnp.maximum(v, 0.0)
                        s = lax.bitcast_convert_type(q, jnp.int32)
                        buf[r, sl] = jnp.where(s >= tvec, q, 0.0)
                    return 0

                lax.fori_loop(0, d // 128, chunk, 0)
            pltpu.async_copy(buf, dst(g), sem_out)

        def group_body(g, _):
            @pl.when(g % 2 == 0)
            def _():
                step(g, buf_a, ia, ib, oa, ob, buf_b)

            @pl.when(g % 2 == 1)
            def _():
                step(g, buf_b, ib, ia, ob, oa, buf_a)

            return 0

        lax.fori_loop(0, groups, group_body, 0)
        pltpu.make_async_copy(buf_a, dst(groups - 2), oa).wait()
        pltpu.make_async_copy(buf_b, dst(groups - 1), ob).wait()

    return zkern(pre, t)


def _dec_body(pre_ref, t_ref, w_ref, bd_ref, out_ref):
    l = pl.program_id(1)
    q = jnp.maximum(pre_ref[...], 0.0)
    s = jax.lax.bitcast_convert_type(q, jnp.int32)
    zb = jnp.where(s >= t_ref[...][:, :1], q, 0.0)
    acc = jax.lax.dot_general(
        zb.astype(jnp.bfloat16), w_ref[...],
        dimension_numbers=(((1,), (0,)), ((), ())),
        preferred_element_type=jnp.float32,
    )

    @pl.when(l == 0)
    def _():
        out_ref[...] = acc + bd_ref[...][None, :]

    @pl.when(l > 0)
    def _():
        out_ref[...] += acc


def _decode(pre, t, W_bf16, b_dec, br, lt):
    n, d_lat = pre.shape
    d_in = W_bf16.shape[1]
    grid = (n // br, d_lat // lt)
    return pl.pallas_call(
        _dec_body,
        grid=grid,
        in_specs=[
            pl.BlockSpec((br, lt), lambda i, l: (i, l)),
            pl.BlockSpec((br, 16), lambda i, l: (i, 0)),
            pl.BlockSpec((lt, d_in), lambda i, l: (l, 0)),
            pl.BlockSpec((d_in,), lambda i, l: (0,)),
        ],
        out_specs=pl.BlockSpec((br, d_in), lambda i, l: (i, 0)),
        out_shape=jax.ShapeDtypeStruct((n, d_in), jnp.float32),
    )(pre, t, W_bf16, b_dec)


@functools.partial(jax.jit, static_argnames=())
def kernel(x, W, b_enc, b_dec):
    W_bf16 = W.astype(jnp.bfloat16)
    pre = _encode(x.astype(jnp.bfloat16), W_bf16, b_enc, br=256, bl=1024)
    t = _thresholds(pre, br=128)
    z = _sc_z(pre, t)
    recon = _decode(pre, t, W_bf16, b_dec, br=512, lt=1024)
    return (recon, z)


# speedup vs baseline: 1.0136x; 1.0136x over previous
import functools

import jax
import jax.numpy as jnp
from jax import lax
from jax.experimental import pallas as pl
from jax.experimental.pallas import tpu as pltpu
from jax.experimental.pallas import tpu_sc as plsc

K = 64


def _enc_body(x_ref, w_ref, be_ref, out_ref):
    acc = jax.lax.dot_general(
        x_ref[...], w_ref[...],
        dimension_numbers=(((1,), (1,)), ((), ())),
        preferred_element_type=jnp.float32,
    )
    out_ref[...] = acc + be_ref[...][None, :]


def _encode(x, W, b_enc, br, bl):
    n, d_in = x.shape
    d_lat = W.shape[0]
    grid = (d_lat // bl, n // br)
    return pl.pallas_call(
        _enc_body,
        grid=grid,
        in_specs=[
            pl.BlockSpec((br, d_in), lambda i, j: (j, 0)),
            pl.BlockSpec((bl, d_in), lambda i, j: (i, 0)),
            pl.BlockSpec((bl,), lambda i, j: (i,)),
        ],
        out_specs=pl.BlockSpec((br, bl), lambda i, j: (j, i)),
        out_shape=jax.ShapeDtypeStruct((n, d_lat), jnp.float32),
    )(x, W, b_enc)


def _thresh_body(pre_ref, t_ref):
    q = jnp.maximum(pre_ref[...], 0.0)
    s = jax.lax.bitcast_convert_type(q, jnp.int32)
    t = jnp.zeros((q.shape[0], 1), dtype=jnp.int32)
    for b in range(30, -1, -1):
        cand = t | (1 << b)
        cnt = jnp.sum(s >= cand, axis=1, keepdims=True, dtype=jnp.int32)
        t = jnp.where(cnt >= K, cand, t)
    t_ref[...] = jnp.broadcast_to(jnp.maximum(t, 1), (t.shape[0], 16))


def _thresholds(pre, br):
    n, d_lat = pre.shape
    return pl.pallas_call(
        _thresh_body,
        grid=(n // br,),
        in_specs=[pl.BlockSpec((br, d_lat), lambda i: (i, 0))],
        out_specs=pl.BlockSpec((br, 16), lambda i: (i, 0)),
        out_shape=jax.ShapeDtypeStruct((n, 16), jnp.int32),
    )(pre)


_GROUP = 4


def _sc_z(pre, t):
    n, d = pre.shape
    nw = 32
    rows_per_w = n // nw
    groups = rows_per_w // _GROUP
    mesh = plsc.VectorSubcoreMesh(core_axis_name="c", subcore_axis_name="s")

    @functools.partial(
        pl.kernel,
        mesh=mesh,
        out_type=jax.ShapeDtypeStruct((n, d), jnp.float32),
        scratch_types=[
            pltpu.VMEM((_GROUP, d), jnp.float32),
            pltpu.VMEM((rows_per_w, 16), jnp.int32),
        ],
    )
    def zkern(pre_hbm, t_hbm, z_hbm, buf_v, t_v):
        cid = lax.axis_index("c")
        sid = lax.axis_index("s")
        wid = sid * 2 + cid
        base = wid * rows_per_w
        pltpu.sync_copy(t_hbm.at[pl.ds(base, rows_per_w)], t_v)

        def group_body(g, _):
            row0 = base + g * _GROUP
            pltpu.sync_copy(pre_hbm.at[pl.ds(row0, _GROUP)], buf_v)
            for r in range(_GROUP):
                tvec = t_v[g * _GROUP + r, :]

                def chunk(j, _, r=r, tvec=tvec):
                    for u in range(8):
                        sl = pl.ds(j * 128 + u * 16, 16)
                        v = buf_v[r, sl]
                        q = jnp.maximum(v, 0.0)
                        s = lax.bitcast_convert_type(q, jnp.int32)
                        buf_v[r, sl] = jnp.where(s >= tvec, q, 0.0)
                    return 0

                lax.fori_loop(0, d // 128, chunk, 0)
            pltpu.sync_copy(buf_v, z_hbm.at[pl.ds(row0, _GROUP)])
            return 0

        lax.fori_loop(0, groups, group_body, 0)

    return zkern(pre, t)


def _dec_body(pre_ref, t_ref, w_ref, bd_ref, out_ref):
    l = pl.program_id(1)
    q = jnp.maximum(pre_ref[...], 0.0)
    s = jax.lax.bitcast_convert_type(q, jnp.int32)
    zb = jnp.where(s >= t_ref[...][:, :1], q, 0.0)
    acc = jax.lax.dot_general(
        zb.astype(jnp.bfloat16), w_ref[...],
        dimension_numbers=(((1,), (0,)), ((), ())),
        preferred_element_type=jnp.float32,
    )

    @pl.when(l == 0)
    def _():
        out_ref[...] = acc + bd_ref[...][None, :]

    @pl.when(l > 0)
    def _():
        out_ref[...] += acc


def _decode(pre, t, W_bf16, b_dec, br, lt):
    n, d_lat = pre.shape
    d_in = W_bf16.shape[1]
    grid = (n // br, d_lat // lt)
    return pl.pallas_call(
        _dec_body,
        grid=grid,
        in_specs=[
            pl.BlockSpec((br, lt), lambda i, l: (i, l)),
            pl.BlockSpec((br, 16), lambda i, l: (i, 0)),
            pl.BlockSpec((lt, d_in), lambda i, l: (l, 0)),
            pl.BlockSpec((d_in,), lambda i, l: (0,)),
        ],
        out_specs=pl.BlockSpec((br, d_in), lambda i, l: (i, 0)),
        out_shape=jax.ShapeDtypeStruct((n, d_in), jnp.float32),
    )(pre, t, W_bf16, b_dec)


@functools.partial(jax.jit, static_argnames=())
def kernel(x, W, b_enc, b_dec):
    W_bf16 = W.astype(jnp.bfloat16)
    pre = _encode(x.astype(jnp.bfloat16), W_bf16, b_enc, br=256, bl=1024)
    t = _thresholds(pre, br=128)
    z = _sc_z(pre, t)
    recon = _decode(pre, t, W_bf16, b_dec, br=512, lt=1024)
    return (recon, z)
